# Initial kernel scaffold; baseline (speedup 1.0000x reference)
import jax
import jax.numpy as jnp
from jax.experimental import pallas as pl


def kernel(x_complete, Wg, bg, Ws, bs, Wp, bp):
    raise NotImplementedError("write your pallas kernel here")


# speedup vs baseline: 2.0351x; 2.0351x over previous
import numpy as np
import jax
import jax.numpy as jnp
from jax.experimental import pallas as pl

WS = 16
K = 32
P = 8
NH = 4
C = 128
H = 128
W = 128
HD = C // NH
NWH = H // WS
NWIN = NWH * NWH
SCALE = HD ** -0.5


def _rope_tables(n, d):
    inv = 1.0 / (10000.0 ** (np.arange(0, d, 2, dtype=np.float32) / np.float32(d)))
    t = np.arange(n, dtype=np.float32)
    freqs = np.repeat(t[:, None] * inv[None, :], 2, axis=1)
    return np.cos(freqs).astype(np.float32), np.sin(freqs).astype(np.float32)


def _rot_matrix(d):
    r = np.zeros((d, d), dtype=np.float32)
    for i in range(d // 2):
        r[2 * i + 1, 2 * i] = -1.0
        r[2 * i, 2 * i + 1] = 1.0
    return r


_CQ, _SQ = _rope_tables(WS * WS, HD)
_CK, _SK = _rope_tables(K * K, HD)
_ROT = _rot_matrix(HD)


def _proj_kernel(xf_ref, wgt_ref, wst_ref, bg_ref, bs_ref, y_ref, xs_ref):
    x = xf_ref[:]
    y_ref[:] = jnp.dot(x, wgt_ref[:], preferred_element_type=jnp.float32) + bg_ref[:]
    xs_ref[:] = jnp.dot(x, wst_ref[:], preferred_element_type=jnp.float32) + bs_ref[:]


def _attn_kernel(y_ref, xa_ref, xb_ref, cq_ref, sq_ref, ck_ref, sk_ref,
                 rot_ref, wpt_ref, bp_ref, out_ref):
    j = pl.program_id(1)
    yw = y_ref[:].reshape(WS * WS, C)
    a = xa_ref[:, pl.ds(j * WS, K), :]
    b = xb_ref[:, pl.ds(j * WS, K), :]
    patch = jnp.concatenate([a, b], axis=0).reshape(K * K, 2 * C)
    rot = rot_ref[:]
    cq, sq, ck, sk = cq_ref[:], sq_ref[:], ck_ref[:], sk_ref[:]
    outs = []
    for h in range(NH):
        qh = yw[:, h * HD:(h + 1) * HD]
        kh = patch[:, h * HD:(h + 1) * HD]
        vh = patch[:, C + h * HD:C + (h + 1) * HD]
        qi = SCALE / jnp.maximum(
            jnp.sqrt(jnp.sum(qh * qh, axis=1, keepdims=True)), 1e-12)
        ki = 1.0 / jnp.maximum(
            jnp.sqrt(jnp.sum(kh * kh, axis=1, keepdims=True)), 1e-12)
        qr = (qh * cq + jnp.dot(qh, rot, preferred_element_type=jnp.float32) * sq) * qi
        kr = (kh * ck + jnp.dot(kh, rot, preferred_element_type=jnp.float32) * sk) * ki
        sim = jax.lax.dot_general(qr, kr, (((1,), (1,)), ((), ())),
                                  preferred_element_type=jnp.float32)
        e = jnp.exp(sim)
        p = e / jnp.sum(e, axis=1, keepdims=True)
        outs.append(jnp.dot(p, vh, preferred_element_type=jnp.float32))
    o = jnp.concatenate(outs, axis=1)
    out_ref[0] = jnp.dot(o, wpt_ref[:], preferred_element_type=jnp.float32) + bp_ref[:]


def kernel(x_complete, Wg, bg, Ws, bs, Wp, bp):
    x = x_complete[0].transpose(1, 2, 0)
    x_flat = x.reshape(H * W, C)

    RB = 1024
    y_flat, xs_flat = pl.pallas_call(
        _proj_kernel,
        grid=(H * W // RB,),
        in_specs=[
            pl.BlockSpec((RB, C), lambda r: (r, 0)),
            pl.BlockSpec((C, C), lambda r: (0, 0)),
            pl.BlockSpec((C, 2 * C), lambda r: (0, 0)),
            pl.BlockSpec((1, C), lambda r: (0, 0)),
            pl.BlockSpec((1, 2 * C), lambda r: (0, 0)),
        ],
        out_specs=[
            pl.BlockSpec((RB, C), lambda r: (r, 0)),
            pl.BlockSpec((RB, 2 * C), lambda r: (r, 0)),
        ],
        out_shape=[
            jax.ShapeDtypeStruct((H * W, C), jnp.float32),
            jax.ShapeDtypeStruct((H * W, 2 * C), jnp.float32),
        ],
    )(x_flat, Wg.T, Ws.T, bg.reshape(1, C), bs.reshape(1, 2 * C))

    y_img = y_flat.reshape(H, W, C)
    xs_pad = jnp.pad(xs_flat.reshape(H, W, 2 * C), ((P, P), (P, P), (0, 0)))

    cq = jnp.asarray(_CQ)
    sq = jnp.asarray(_SQ)
    ck = jnp.asarray(_CK)
    sk = jnp.asarray(_SK)
    rot = jnp.asarray(_ROT)

    out = pl.pallas_call(
        _attn_kernel,
        grid=(NWH, NWH),
        in_specs=[
            pl.BlockSpec((WS, WS, C), lambda i, j: (i, j, 0)),
            pl.BlockSpec((WS, W + 2 * P, 2 * C), lambda i, j: (i, 0, 0)),
            pl.BlockSpec((WS, W + 2 * P, 2 * C), lambda i, j: (i + 1, 0, 0)),
            pl.BlockSpec((WS * WS, HD), lambda i, j: (0, 0)),
            pl.BlockSpec((WS * WS, HD), lambda i, j: (0, 0)),
            pl.BlockSpec((K * K, HD), lambda i, j: (0, 0)),
            pl.BlockSpec((K * K, HD), lambda i, j: (0, 0)),
            pl.BlockSpec((HD, HD), lambda i, j: (0, 0)),
            pl.BlockSpec((C, C), lambda i, j: (0, 0)),
            pl.BlockSpec((1, C), lambda i, j: (0, 0)),
        ],
        out_specs=pl.BlockSpec((1, WS * WS, C), lambda i, j: (i * NWH + j, 0, 0)),
        out_shape=jax.ShapeDtypeStruct((NWIN, WS * WS, C), jnp.float32),
    )(y_img, xs_pad, xs_pad, cq, sq, ck, sk, rot, Wp.T, bp.reshape(1, C))

    return out


# speedup vs baseline: 4.4983x; 2.2104x over previous
import numpy as np
import jax
import jax.numpy as jnp
from jax.experimental import pallas as pl

WS = 16
K = 32
P = 8
NH = 4
C = 128
H = 128
W = 128
HD = C // NH
NWH = H // WS
NWIN = NWH * NWH
NQ = WS * WS
NK = K * K
SCALE = HD ** -0.5
F32 = jnp.float32


def _rope_tables(n, d):
    inv = 1.0 / (10000.0 ** (np.arange(0, d, 2, dtype=np.float32) / np.float32(d)))
    t = np.arange(n, dtype=np.float32)
    freqs = np.repeat(t[:, None] * inv[None, :], 2, axis=1)
    return np.cos(freqs).astype(np.float32), np.sin(freqs).astype(np.float32)


def _rot_matrix(d):
    r = np.zeros((d, d), dtype=np.float32)
    for i in range(d // 2):
        r[2 * i + 1, 2 * i] = -1.0
        r[2 * i, 2 * i + 1] = 1.0
    return r


def _blockdiag(block, n):
    d = block.shape[0]
    out = np.zeros((d * n, d * n), dtype=np.float32)
    for i in range(n):
        out[i * d:(i + 1) * d, i * d:(i + 1) * d] = block
    return out


_cq, _sq = _rope_tables(NQ, HD)
_ck, _sk = _rope_tables(NK, HD)
_CQ = np.tile(_cq, (1, NH))
_SQ = np.tile(_sq, (1, NH))
_CKF = np.tile(_ck, (1, NH))
_SKF = np.tile(_sk, (1, NH))
_ROTF = _blockdiag(_rot_matrix(HD), NH)
_BD = _blockdiag(np.ones((HD, HD), np.float32), NH)


def _proj_kernel(xf_ref, wgt_ref, wst_ref, bg_ref, bs_ref, y_ref, xs_ref):
    x = xf_ref[:]
    y_ref[:] = jnp.dot(x, wgt_ref[:], preferred_element_type=F32) + bg_ref[:]
    xs_ref[:] = jnp.dot(x, wst_ref[:], preferred_element_type=F32) + bs_ref[:]


def _rope_norm(x, cos, sin, rotf, bd, scale):
    n = jnp.dot(x * x, bd, preferred_element_type=F32)
    inv = scale * jax.lax.rsqrt(jnp.maximum(n, 1e-24))
    return (x * cos + jnp.dot(x, rotf, preferred_element_type=F32) * sin) * inv


def _attn_kernel(y_ref, xa_ref, xb_ref, cq_ref, sq_ref, cka_ref, ska_ref,
                 ckb_ref, skb_ref, rotf_ref, bd_ref, wpt_ref, bp_ref, out_ref):
    j = pl.program_id(1)
    yw = y_ref[:].reshape(NQ, C)
    a2 = xa_ref[:, pl.ds(j * WS, K), :].reshape(NK // 2, 2 * C)
    b2 = xb_ref[:, pl.ds(j * WS, K), :].reshape(NK // 2, 2 * C)
    ka, va = a2[:, :C], a2[:, C:]
    kb, vb = b2[:, :C], b2[:, C:]
    rotf = rotf_ref[:]
    bd = bd_ref[:]

    qr = _rope_norm(yw, cq_ref[:], sq_ref[:], rotf, bd, SCALE)
    kra = _rope_norm(ka, cka_ref[:], ska_ref[:], rotf, bd, 1.0)
    krb = _rope_norm(kb, ckb_ref[:], skb_ref[:], rotf, bd, 1.0)

    acc = jnp.zeros((NQ, C), dtype=F32)
    for h in range(NH):
        sl = slice(h * HD, (h + 1) * HD)
        sim_a = jax.lax.dot_general(qr[:, sl], kra[:, sl], (((1,), (1,)), ((), ())),
                                    preferred_element_type=F32)
        sim_b = jax.lax.dot_general(qr[:, sl], krb[:, sl], (((1,), (1,)), ((), ())),
                                    preferred_element_type=F32)
        ea = jnp.exp(sim_a)
        eb = jnp.exp(sim_b)
        s = jnp.sum(ea, axis=1, keepdims=True) + jnp.sum(eb, axis=1, keepdims=True)
        num = (jnp.dot(ea, va[:, sl], preferred_element_type=F32)
               + jnp.dot(eb, vb[:, sl], preferred_element_type=F32))
        oh = num * (1.0 / s)
        acc = acc + jnp.dot(oh, wpt_ref[sl, :], preferred_element_type=F32)
    out_ref[0] = acc + bp_ref[:]


def kernel(x_complete, Wg, bg, Ws, bs, Wp, bp):
    x = x_complete[0].transpose(1, 2, 0)
    x_flat = x.reshape(H * W, C)

    RB = 1024
    y_flat, xs_flat = pl.pallas_call(
        _proj_kernel,
        grid=(H * W // RB,),
        in_specs=[
            pl.BlockSpec((RB, C), lambda r: (r, 0)),
            pl.BlockSpec((C, C), lambda r: (0, 0)),
            pl.BlockSpec((C, 2 * C), lambda r: (0, 0)),
            pl.BlockSpec((1, C), lambda r: (0, 0)),
            pl.BlockSpec((1, 2 * C), lambda r: (0, 0)),
        ],
        out_specs=[
            pl.BlockSpec((RB, C), lambda r: (r, 0)),
            pl.BlockSpec((RB, 2 * C), lambda r: (r, 0)),
        ],
        out_shape=[
            jax.ShapeDtypeStruct((H * W, C), F32),
            jax.ShapeDtypeStruct((H * W, 2 * C), F32),
        ],
    )(x_flat, Wg.T, Ws.T, bg.reshape(1, C), bs.reshape(1, 2 * C))

    y_img = y_flat.reshape(H, W, C)
    xs_pad = jnp.pad(xs_flat.reshape(H, W, 2 * C), ((P, P), (P, P), (0, 0)))

    out = pl.pallas_call(
        _attn_kernel,
        grid=(NWH, NWH),
        in_specs=[
            pl.BlockSpec((WS, WS, C), lambda i, j: (i, j, 0)),
            pl.BlockSpec((WS, W + 2 * P, 2 * C), lambda i, j: (i, 0, 0)),
            pl.BlockSpec((WS, W + 2 * P, 2 * C), lambda i, j: (i + 1, 0, 0)),
            pl.BlockSpec((NQ, C), lambda i, j: (0, 0)),
            pl.BlockSpec((NQ, C), lambda i, j: (0, 0)),
            pl.BlockSpec((NK // 2, C), lambda i, j: (0, 0)),
            pl.BlockSpec((NK // 2, C), lambda i, j: (0, 0)),
            pl.BlockSpec((NK // 2, C), lambda i, j: (0, 0)),
            pl.BlockSpec((NK // 2, C), lambda i, j: (0, 0)),
            pl.BlockSpec((C, C), lambda i, j: (0, 0)),
            pl.BlockSpec((C, C), lambda i, j: (0, 0)),
            pl.BlockSpec((C, C), lambda i, j: (0, 0)),
            pl.BlockSpec((1, C), lambda i, j: (0, 0)),
        ],
        out_specs=pl.BlockSpec((1, NQ, C), lambda i, j: (i * NWH + j, 0, 0)),
        out_shape=jax.ShapeDtypeStruct((NWIN, NQ, C), F32),
    )(y_img, xs_pad, xs_pad,
      jnp.asarray(_CQ), jnp.asarray(_SQ),
      jnp.asarray(_CKF[:NK // 2]), jnp.asarray(_SKF[:NK // 2]),
      jnp.asarray(_CKF[NK // 2:]), jnp.asarray(_SKF[NK // 2:]),
      jnp.asarray(_ROTF), jnp.asarray(_BD),
      Wp.T, bp.reshape(1, C))

    return out


# speedup vs baseline: 4.7287x; 1.0512x over previous
import numpy as np
import jax
import jax.numpy as jnp
from jax.experimental import pallas as pl

WS = 16
K = 32
P = 8
NH = 4
C = 128
H = 128
W = 128
HD = C // NH
NWH = H // WS
NWIN = NWH * NWH
NQ = WS * WS
NK = K * K
SCALE = HD ** -0.5
F32 = jnp.float32


def _rope_tables(n, d):
    inv = 1.0 / (10000.0 ** (np.arange(0, d, 2, dtype=np.float32) / np.float32(d)))
    t = np.arange(n, dtype=np.float32)
    freqs = np.repeat(t[:, None] * inv[None, :], 2, axis=1)
    return np.cos(freqs).astype(np.float32), np.sin(freqs).astype(np.float32)


def _rot_matrix(d):
    r = np.zeros((d, d), dtype=np.float32)
    for i in range(d // 2):
        r[2 * i + 1, 2 * i] = -1.0
        r[2 * i, 2 * i + 1] = 1.0
    return r


def _blockdiag(block, n):
    d = block.shape[0]
    out = np.zeros((d * n, d * n), dtype=np.float32)
    for i in range(n):
        out[i * d:(i + 1) * d, i * d:(i + 1) * d] = block
    return out


_cq, _sq = _rope_tables(NQ, HD)
_ck, _sk = _rope_tables(NK, HD)
_CQ = np.tile(_cq, (1, NH))
_SQ = np.tile(_sq, (1, NH))
_CKF = np.tile(_ck, (1, NH))
_SKF = np.tile(_sk, (1, NH))
_ROTF = _blockdiag(_rot_matrix(HD), NH)
_BD = _blockdiag(np.ones((HD, HD), np.float32), NH)


def _proj_kernel(xf_ref, wgt_ref, wst_ref, bg_ref, bs_ref, y_ref, xs_ref):
    s = pl.program_id(0)
    interior = jnp.logical_and(s >= 1, s <= 16)

    @pl.when(interior)
    def _():
        xt = xf_ref[:].T
        y_ref[:] = jnp.dot(xt, wgt_ref[:], preferred_element_type=F32) + bg_ref[:]
        band = jnp.dot(xt, wst_ref[:], preferred_element_type=F32) + bs_ref[:]
        xs_ref[:, :P, :] = jnp.zeros((P, P, 2 * C), F32)
        xs_ref[:, P:P + W, :] = band.reshape(P, W, 2 * C)
        xs_ref[:, P + W:, :] = jnp.zeros((P, P, 2 * C), F32)

    @pl.when(jnp.logical_not(interior))
    def _():
        xs_ref[:] = jnp.zeros((P, W + 2 * P, 2 * C), F32)


def _rope_norm(x, cos, sin, rotf, bd, scale):
    n = jnp.dot(x * x, bd, preferred_element_type=F32)
    inv = scale * jax.lax.rsqrt(jnp.maximum(n, 1e-24))
    return (x * cos + jnp.dot(x, rotf, preferred_element_type=F32) * sin) * inv


def _attn_kernel(y_ref, xa_ref, xb_ref, cq_ref, sq_ref, cka_ref, ska_ref,
                 ckb_ref, skb_ref, rotf_ref, bd_ref, wpt_ref, bp_ref, out_ref):
    j = pl.program_id(1)
    yw = y_ref[:].reshape(NQ, C)
    a2 = xa_ref[:, pl.ds(j * WS, K), :].reshape(NK // 2, 2 * C)
    b2 = xb_ref[:, pl.ds(j * WS, K), :].reshape(NK // 2, 2 * C)
    ka, va = a2[:, :C], a2[:, C:]
    kb, vb = b2[:, :C], b2[:, C:]
    rotf = rotf_ref[:]
    bd = bd_ref[:]

    qr = _rope_norm(yw, cq_ref[:], sq_ref[:], rotf, bd, SCALE)
    kra = _rope_norm(ka, cka_ref[:], ska_ref[:], rotf, bd, 1.0)
    krb = _rope_norm(kb, ckb_ref[:], skb_ref[:], rotf, bd, 1.0)

    acc = jnp.zeros((NQ, C), dtype=F32)
    for h in range(NH):
        sl = slice(h * HD, (h + 1) * HD)
        sim_a = jax.lax.dot_general(qr[:, sl], kra[:, sl], (((1,), (1,)), ((), ())),
                                    preferred_element_type=F32)
        sim_b = jax.lax.dot_general(qr[:, sl], krb[:, sl], (((1,), (1,)), ((), ())),
                                    preferred_element_type=F32)
        ea = jnp.exp(sim_a)
        eb = jnp.exp(sim_b)
        s = jnp.sum(ea, axis=1, keepdims=True) + jnp.sum(eb, axis=1, keepdims=True)
        num = (jnp.dot(ea, va[:, sl], preferred_element_type=F32)
               + jnp.dot(eb, vb[:, sl], preferred_element_type=F32))
        oh = num * (1.0 / s)
        acc = acc + jnp.dot(oh, wpt_ref[sl, :], preferred_element_type=F32)
    out_ref[0] = acc + bp_ref[:]


def kernel(x_complete, Wg, bg, Ws, bs, Wp, bp):
    x2d = x_complete.reshape(C, H * W)

    RB = 1024
    nbands = H * W // RB
    y_flat, xs_pad = pl.pallas_call(
        _proj_kernel,
        grid=(nbands + 2,),
        in_specs=[
            pl.BlockSpec((C, RB), lambda s: (0, jnp.clip(s - 1, 0, nbands - 1))),
            pl.BlockSpec((C, C), lambda s: (0, 0)),
            pl.BlockSpec((C, 2 * C), lambda s: (0, 0)),
            pl.BlockSpec((1, C), lambda s: (0, 0)),
            pl.BlockSpec((1, 2 * C), lambda s: (0, 0)),
        ],
        out_specs=[
            pl.BlockSpec((RB, C), lambda s: (jnp.clip(s - 1, 0, nbands - 1), 0)),
            pl.BlockSpec((P, W + 2 * P, 2 * C), lambda s: (s, 0, 0)),
        ],
        out_shape=[
            jax.ShapeDtypeStruct((H * W, C), F32),
            jax.ShapeDtypeStruct((H + 2 * P, W + 2 * P, 2 * C), F32),
        ],
    )(x2d, Wg.T, Ws.T, bg.reshape(1, C), bs.reshape(1, 2 * C))

    y_img = y_flat.reshape(H, W, C)

    out = pl.pallas_call(
        _attn_kernel,
        grid=(NWH, NWH),
        in_specs=[
            pl.BlockSpec((WS, WS, C), lambda i, j: (i, j, 0)),
            pl.BlockSpec((WS, W + 2 * P, 2 * C), lambda i, j: (i, 0, 0)),
            pl.BlockSpec((WS, W + 2 * P, 2 * C), lambda i, j: (i + 1, 0, 0)),
            pl.BlockSpec((NQ, C), lambda i, j: (0, 0)),
            pl.BlockSpec((NQ, C), lambda i, j: (0, 0)),
            pl.BlockSpec((NK // 2, C), lambda i, j: (0, 0)),
            pl.BlockSpec((NK // 2, C), lambda i, j: (0, 0)),
            pl.BlockSpec((NK // 2, C), lambda i, j: (0, 0)),
            pl.BlockSpec((NK // 2, C), lambda i, j: (0, 0)),
            pl.BlockSpec((C, C), lambda i, j: (0, 0)),
            pl.BlockSpec((C, C), lambda i, j: (0, 0)),
            pl.BlockSpec((C, C), lambda i, j: (0, 0)),
            pl.BlockSpec((1, C), lambda i, j: (0, 0)),
        ],
        out_specs=pl.BlockSpec((1, NQ, C), lambda i, j: (i * NWH + j, 0, 0)),
        out_shape=jax.ShapeDtypeStruct((NWIN, NQ, C), F32),
    )(y_img, xs_pad, xs_pad,
      jnp.asarray(_CQ), jnp.asarray(_SQ),
      jnp.asarray(_CKF[:NK // 2]), jnp.asarray(_SKF[:NK // 2]),
      jnp.asarray(_CKF[NK // 2:]), jnp.asarray(_SKF[NK // 2:]),
      jnp.asarray(_ROTF), jnp.asarray(_BD),
      Wp.T, bp.reshape(1, C))

    return out


# speedup vs baseline: 5.1600x; 1.0912x over previous
import numpy as np
import jax
import jax.numpy as jnp
from jax.experimental import pallas as pl

WS = 16
K = 32
P = 8
NH = 4
C = 128
H = 128
W = 128
HD = C // NH
NWH = H // WS
NWIN = NWH * NWH
NQ = WS * WS
NK = K * K
SCALE = HD ** -0.5
F32 = jnp.float32
BF16 = jnp.bfloat16


def _rope_tables(n, d):
    inv = 1.0 / (10000.0 ** (np.arange(0, d, 2, dtype=np.float32) / np.float32(d)))
    t = np.arange(n, dtype=np.float32)
    freqs = np.repeat(t[:, None] * inv[None, :], 2, axis=1)
    return np.cos(freqs).astype(np.float32), np.sin(freqs).astype(np.float32)


def _rot_matrix(d):
    r = np.zeros((d, d), dtype=np.float32)
    for i in range(d // 2):
        r[2 * i + 1, 2 * i] = -1.0
        r[2 * i, 2 * i + 1] = 1.0
    return r


def _blockdiag(block, n):
    d = block.shape[0]
    out = np.zeros((d * n, d * n), dtype=np.float32)
    for i in range(n):
        out[i * d:(i + 1) * d, i * d:(i + 1) * d] = block
    return out


_cq, _sq = _rope_tables(NQ, HD)
_ck, _sk = _rope_tables(NK, HD)
_CQ = np.tile(_cq, (1, NH))
_SQ = np.tile(_sq, (1, NH))
_CKF = np.tile(_ck, (1, NH))
_SKF = np.tile(_sk, (1, NH))
_ROTF = _blockdiag(_rot_matrix(HD), NH)
_BD = _blockdiag(np.ones((HD, HD), np.float32), NH)


def _proj_kernel(xf_ref, wgt_ref, wst_ref, bg_ref, bs_ref, y_ref, xs_ref):
    s = pl.program_id(0)
    interior = jnp.logical_and(s >= 1, s <= 16)

    @pl.when(interior)
    def _():
        xt = xf_ref[:].T
        y_ref[:] = jnp.dot(xt, wgt_ref[:], preferred_element_type=F32) + bg_ref[:]
        band = jnp.dot(xt, wst_ref[:], preferred_element_type=F32) + bs_ref[:]
        z = jnp.zeros((P, P, 2 * C), BF16)
        xs_ref[:] = jnp.concatenate(
            [z, band.reshape(P, W, 2 * C).astype(BF16), z], axis=1)

    @pl.when(jnp.logical_not(interior))
    def _():
        xs_ref[:] = jnp.zeros((P, W + 2 * P, 2 * C), BF16)


def _rope_norm(x, cos, sin, rotf, bd, scale):
    n = jnp.dot(x * x, bd, preferred_element_type=F32)
    inv = scale * jax.lax.rsqrt(jnp.maximum(n, 1e-24))
    return (x * cos + jnp.dot(x, rotf, preferred_element_type=F32) * sin) * inv


def _attn_kernel(y_ref, xa_ref, xb_ref, cq_ref, sq_ref, cka_ref, ska_ref,
                 ckb_ref, skb_ref, rotf_ref, bd_ref, wpt_ref, bp_ref, out_ref):
    j = pl.program_id(1)
    yw = y_ref[:].reshape(NQ, C)
    a2 = xa_ref[:, pl.ds(j * WS, K), :].reshape(NK // 2, 2 * C)
    b2 = xb_ref[:, pl.ds(j * WS, K), :].reshape(NK // 2, 2 * C)
    ka, va = a2[:, :C].astype(F32), a2[:, C:]
    kb, vb = b2[:, :C].astype(F32), b2[:, C:]
    rotf = rotf_ref[:]
    bd = bd_ref[:]

    qr = _rope_norm(yw, cq_ref[:], sq_ref[:], rotf, bd, SCALE).astype(BF16)
    kra = _rope_norm(ka, cka_ref[:], ska_ref[:], rotf, bd, 1.0).astype(BF16)
    krb = _rope_norm(kb, ckb_ref[:], skb_ref[:], rotf, bd, 1.0).astype(BF16)

    acc = jnp.zeros((NQ, C), dtype=F32)
    for h in range(NH):
        sl = slice(h * HD, (h + 1) * HD)
        sim_a = jax.lax.dot_general(qr[:, sl], kra[:, sl], (((1,), (1,)), ((), ())),
                                    preferred_element_type=F32)
        sim_b = jax.lax.dot_general(qr[:, sl], krb[:, sl], (((1,), (1,)), ((), ())),
                                    preferred_element_type=F32)
        ea_f = jnp.exp(sim_a)
        eb_f = jnp.exp(sim_b)
        ea = ea_f.astype(BF16)
        eb = eb_f.astype(BF16)
        s = (jnp.sum(ea_f, axis=1, keepdims=True)
             + jnp.sum(eb_f, axis=1, keepdims=True))
        num = (jnp.dot(ea, va[:, sl], preferred_element_type=F32)
               + jnp.dot(eb, vb[:, sl], preferred_element_type=F32))
        oh = num * (1.0 / s)
        acc = acc + jnp.dot(oh, wpt_ref[sl, :], preferred_element_type=F32)
    out_ref[0] = acc + bp_ref[:]


def kernel(x_complete, Wg, bg, Ws, bs, Wp, bp):
    x2d = x_complete.reshape(C, H * W)

    RB = 1024
    nbands = H * W // RB
    y_flat, xs_pad = pl.pallas_call(
        _proj_kernel,
        grid=(nbands + 2,),
        in_specs=[
            pl.BlockSpec((C, RB), lambda s: (0, jnp.clip(s - 1, 0, nbands - 1))),
            pl.BlockSpec((C, C), lambda s: (0, 0)),
            pl.BlockSpec((C, 2 * C), lambda s: (0, 0)),
            pl.BlockSpec((1, C), lambda s: (0, 0)),
            pl.BlockSpec((1, 2 * C), lambda s: (0, 0)),
        ],
        out_specs=[
            pl.BlockSpec((RB, C), lambda s: (jnp.clip(s - 1, 0, nbands - 1), 0)),
            pl.BlockSpec((P, W + 2 * P, 2 * C), lambda s: (s, 0, 0)),
        ],
        out_shape=[
            jax.ShapeDtypeStruct((H * W, C), F32),
            jax.ShapeDtypeStruct((H + 2 * P, W + 2 * P, 2 * C), BF16),
        ],
    )(x2d, Wg.T, Ws.T, bg.reshape(1, C), bs.reshape(1, 2 * C))

    y_img = y_flat.reshape(H, W, C)

    out = pl.pallas_call(
        _attn_kernel,
        grid=(NWH, NWH),
        in_specs=[
            pl.BlockSpec((WS, WS, C), lambda i, j: (i, j, 0)),
            pl.BlockSpec((WS, W + 2 * P, 2 * C), lambda i, j: (i, 0, 0)),
            pl.BlockSpec((WS, W + 2 * P, 2 * C), lambda i, j: (i + 1, 0, 0)),
            pl.BlockSpec((NQ, C), lambda i, j: (0, 0)),
            pl.BlockSpec((NQ, C), lambda i, j: (0, 0)),
            pl.BlockSpec((NK // 2, C), lambda i, j: (0, 0)),
            pl.BlockSpec((NK // 2, C), lambda i, j: (0, 0)),
            pl.BlockSpec((NK // 2, C), lambda i, j: (0, 0)),
            pl.BlockSpec((NK // 2, C), lambda i, j: (0, 0)),
            pl.BlockSpec((C, C), lambda i, j: (0, 0)),
            pl.BlockSpec((C, C), lambda i, j: (0, 0)),
            pl.BlockSpec((C, C), lambda i, j: (0, 0)),
            pl.BlockSpec((1, C), lambda i, j: (0, 0)),
        ],
        out_specs=pl.BlockSpec((1, NQ, C), lambda i, j: (i * NWH + j, 0, 0)),
        out_shape=jax.ShapeDtypeStruct((NWIN, NQ, C), F32),
    )(y_img, xs_pad, xs_pad,
      jnp.asarray(_CQ), jnp.asarray(_SQ),
      jnp.asarray(_CKF[:NK // 2]), jnp.asarray(_SKF[:NK // 2]),
      jnp.asarray(_CKF[NK // 2:]), jnp.asarray(_SKF[NK // 2:]),
      jnp.asarray(_ROTF), jnp.asarray(_BD),
      Wp.T, bp.reshape(1, C))

    return out
